# Initial kernel scaffold; baseline (speedup 1.0000x reference)
import jax
import jax.numpy as jnp
from jax.experimental import pallas as pl


def kernel(x, edge_index, W1, b1, W2, b2):
    raise NotImplementedError("write your pallas kernel here")


# speedup vs baseline: 23.0893x; 23.0893x over previous
import functools

import jax
import jax.numpy as jnp
from jax import lax
from jax.experimental import pallas as pl
from jax.experimental.pallas import tpu as pltpu
from jax.experimental.pallas import tpu_sc as plsc

N = 10000
E = 320000
D_IN = 128
D_HID = 128
D_OUT = 64

NC = 2
NS = 16
LANES = 16
CHUNK = 128
CPT = 157
EPT = CPT * CHUNK
NPAD = 10016
RPT = N // NS

_mesh = plsc.VectorSubcoreMesh(core_axis_name="c", subcore_axis_name="s")


def _deg_body(dst_hbm, out_hbm, deg_v, dst_v):
    c = lax.axis_index("c")
    s = lax.axis_index("s")

    zeros16 = jnp.zeros((LANES,), jnp.float32)

    def zero_body(i, carry):
        deg_v[pl.ds(i * LANES, LANES)] = zeros16
        return carry

    lax.fori_loop(0, NPAD // LANES, zero_body, 0)

    pltpu.sync_copy(dst_hbm.at[s], dst_v)

    ones16 = jnp.ones((LANES,), jnp.float32)
    half = (CPT + 1) // 2

    def edge_body(j, carry):
        for k in range(CHUNK // LANES):
            idx = dst_v[j, pl.ds(k * LANES, LANES)]
            plsc.addupdate_scatter(deg_v, [idx], ones16)
        return carry

    start = c * half
    stop = jnp.where(c == 0, half, CPT)
    lax.fori_loop(start, stop, edge_body, 0)

    wid = s * NC + c
    pltpu.sync_copy(deg_v, out_hbm.at[wid])


_deg = functools.partial(
    pl.kernel,
    out_type=jax.ShapeDtypeStruct((NC * NS, NPAD), jnp.float32),
    mesh=_mesh,
    scratch_types=[
        pltpu.VMEM((NPAD,), jnp.float32),
        pltpu.VMEM((CPT, CHUNK), jnp.int32),
    ],
    compiler_params=pltpu.CompilerParams(needs_layout_passes=False, use_tc_tiling_on_sc=False),
)(_deg_body)


def _make_agg(dh):
    def agg_body(hs_hbm, src_hbm, dst_hbm, out_hbm,
                 hs_sh, acc_sh, src_v, dst_v, buf, sem):
        c = lax.axis_index("c")
        s = lax.axis_index("s")
        rs = s * RPT

        pltpu.sync_copy(hs_hbm.at[c, pl.ds(rs, RPT)], hs_sh.at[pl.ds(rs, RPT)])
        pltpu.sync_copy(hs_hbm.at[c, pl.ds(rs, RPT)], acc_sh.at[pl.ds(rs, RPT)])
        pltpu.sync_copy(src_hbm.at[s], src_v)
        pltpu.sync_copy(dst_hbm.at[s], dst_v)
        plsc.subcore_barrier()

        def chunk_body(j, carry):
            pltpu.async_copy(hs_sh.at[src_v.at[j]], buf, sem).wait()
            pltpu.sync_copy(buf, acc_sh.at[dst_v.at[j]], add=True)
            return carry

        lax.fori_loop(0, CPT, chunk_body, 0)
        plsc.subcore_barrier()

        pltpu.sync_copy(acc_sh.at[pl.ds(rs, RPT)], out_hbm.at[c, pl.ds(rs, RPT)])

    return functools.partial(
        pl.kernel,
        out_type=jax.ShapeDtypeStruct((NC, N, dh), jnp.float32),
        mesh=_mesh,
        scratch_types=[
            pltpu.VMEM_SHARED((N, dh), jnp.float32),
            pltpu.VMEM_SHARED((NPAD, dh), jnp.float32),
            pltpu.VMEM((CPT, CHUNK), jnp.int32),
            pltpu.VMEM((CPT, CHUNK), jnp.int32),
            pltpu.VMEM((CHUNK, dh), jnp.float32),
            pltpu.SemaphoreType.DMA,
        ],
        compiler_params=pltpu.CompilerParams(needs_layout_passes=False, use_tc_tiling_on_sc=False),
    )(agg_body)


_agg_hid = _make_agg(D_HID // NC)
_agg_out = _make_agg(D_OUT // NC)


def _prep_body(degt_ref, x_ref, w1_ref, hs_ref, dis_ref):
    deg = jnp.sum(degt_ref[...], axis=1, keepdims=True) + 1.0
    dis = lax.rsqrt(deg)
    h = jnp.dot(x_ref[...], w1_ref[...], preferred_element_type=jnp.float32)
    hs = h * dis
    hs_ref[0] = hs[:, : D_HID // 2]
    hs_ref[1] = hs[:, D_HID // 2:]
    dis_ref[...] = dis


def _mid_body(agg_ref, dis_ref, b1_ref, w2_ref, out_ref):
    dis = dis_ref[...]
    h0 = jnp.maximum(agg_ref[0] * dis + b1_ref[0, : D_HID // 2], 0.0)
    h1 = jnp.maximum(agg_ref[1] * dis + b1_ref[0, D_HID // 2:], 0.0)
    hs2 = jnp.dot(h0, w2_ref[: D_HID // 2], preferred_element_type=jnp.float32)
    hs2 = hs2 + jnp.dot(h1, w2_ref[D_HID // 2:], preferred_element_type=jnp.float32)
    hs2 = hs2 * dis
    out_ref[0] = hs2[:, : D_OUT // 2]
    out_ref[1] = hs2[:, D_OUT // 2:]


def _final_body(agg_ref, dis_ref, b2_ref, out_ref):
    dis = dis_ref[...]
    out_ref[:, : D_OUT // 2] = agg_ref[0] * dis + b2_ref[0, : D_OUT // 2]
    out_ref[:, D_OUT // 2:] = agg_ref[1] * dis + b2_ref[0, D_OUT // 2:]


_prep = pl.pallas_call(
    _prep_body,
    out_shape=[
        jax.ShapeDtypeStruct((NC, N, D_HID // 2), jnp.float32),
        jax.ShapeDtypeStruct((N, 1), jnp.float32),
    ],
)

_mid = pl.pallas_call(
    _mid_body,
    out_shape=jax.ShapeDtypeStruct((NC, N, D_OUT // 2), jnp.float32),
)

_final = pl.pallas_call(
    _final_body,
    out_shape=jax.ShapeDtypeStruct((N, D_OUT), jnp.float32),
)


@jax.jit
def kernel(x, edge_index, W1, b1, W2, b2):
    src = edge_index[0].astype(jnp.int32)
    dst = edge_index[1].astype(jnp.int32)
    pad = NS * EPT - E
    src_p = jnp.concatenate([src, jnp.zeros((pad,), jnp.int32)]).reshape(NS, CPT, CHUNK)
    dst_p = jnp.concatenate([dst, jnp.full((pad,), N, jnp.int32)]).reshape(NS, CPT, CHUNK)

    deg_parts = _deg(dst_p)
    degt = deg_parts.T[:N]
    hs1, dis = _prep(degt, x, W1)
    agg1 = _agg_hid(hs1, src_p, dst_p)
    hs2 = _mid(agg1, dis, b1.reshape(1, -1), W2)
    agg2 = _agg_out(hs2, src_p, dst_p)
    return _final(agg2, dis, b2.reshape(1, -1))
